# baseline (speedup 1.0000x reference)
import functools

import jax
import jax.numpy as jnp
from jax import lax
from jax.experimental import pallas as pl
from jax.experimental.pallas import tpu as pltpu
from jax.experimental.pallas import tpu_sc as plsc

NC, NS, L = 2, 16, 16
NW = NC * NS
ALPHA = 0.2
BE = 128

_GATHER_1D = lax.GatherDimensionNumbers(
    offset_dims=(), collapsed_slice_dims=(0,), start_index_map=(0,))


def _bcast_lane(v16, lane):
    idx = jnp.full((L, 1), lane, jnp.int32)
    return lax.gather(v16, idx, _GATHER_1D, (1,),
                      mode=lax.GatherScatterMode.PROMISE_IN_BOUNDS)


def _tc_pre(x, W0, a2d):
    n, f = x.shape
    br = 1000

    def body(x_ref, w_ref, a_ref, h_ref, pq_ref):
        h = jnp.dot(x_ref[...], w_ref[...], preferred_element_type=jnp.float32)
        h_ref[...] = h
        pq_ref[...] = jnp.dot(h, a_ref[...], preferred_element_type=jnp.float32)

    return pl.pallas_call(
        body,
        grid=(n // br,),
        in_specs=[
            pl.BlockSpec((br, f), lambda i: (i, 0)),
            pl.BlockSpec((f, f), lambda i: (0, 0)),
            pl.BlockSpec((f, 2), lambda i: (0, 0)),
        ],
        out_specs=[
            pl.BlockSpec((br, f), lambda i: (i, 0)),
            pl.BlockSpec((br, 2), lambda i: (i, 0)),
        ],
        out_shape=[
            jax.ShapeDtypeStruct((n, f), jnp.float32),
            jax.ShapeDtypeStruct((n, 2), jnp.float32),
        ],
    )(x, W0, a2d)


def _tc_post(hp0, hp1, rec_n, Wp, bp):
    n, f = hp0.shape
    br = 1000

    def body(h0_ref, h1_ref, r_ref, wp_ref, bp_ref, o_ref):
        hp = (h0_ref[...] + h1_ref[...]) * r_ref[...]
        o = lax.dot_general(hp, wp_ref[...], (((1,), (1,)), ((), ())),
                            preferred_element_type=jnp.float32)
        o_ref[...] = o + bp_ref[...]

    return pl.pallas_call(
        body,
        grid=(n // br,),
        in_specs=[
            pl.BlockSpec((br, f), lambda i: (i, 0)),
            pl.BlockSpec((br, f), lambda i: (i, 0)),
            pl.BlockSpec((br, 1), lambda i: (i, 0)),
            pl.BlockSpec((f, f), lambda i: (0, 0)),
            pl.BlockSpec((1, f), lambda i: (0, 0)),
        ],
        out_specs=pl.BlockSpec((br, f), lambda i: (i, 0)),
        out_shape=jax.ShapeDtypeStruct((n, f), jnp.float32),
    )(hp0, hp1, rec_n, Wp, bp.reshape(1, f))


def _sc_pass1(src_p, tgt_p, p_pad, q_pad, *, n_pad, n_true, e_true):
    epad = src_p.shape[0]
    chunk = epad // NW
    nvec = chunk // L
    nslice = n_pad // NS
    mesh = plsc.VectorSubcoreMesh(core_axis_name="c", subcore_axis_name="s")

    @functools.partial(
        pl.kernel,
        out_type=(
            jax.ShapeDtypeStruct((NC, n_pad), jnp.float32),
            jax.ShapeDtypeStruct((NC, n_pad), jnp.float32),
            jax.ShapeDtypeStruct((epad,), jnp.float32),
        ),
        mesh=mesh,
        compiler_params=pltpu.CompilerParams(needs_layout_passes=False),
        scratch_types=(
            pltpu.VMEM((n_pad,), jnp.float32),
            pltpu.VMEM((n_pad,), jnp.float32),
            pltpu.VMEM((chunk,), jnp.int32),
            pltpu.VMEM((chunk,), jnp.int32),
            pltpu.VMEM((chunk,), jnp.float32),
            pltpu.VMEM((n_pad,), jnp.float32),
            pltpu.VMEM((n_pad,), jnp.float32),
            pltpu.VMEM((n_pad // NS,), jnp.float32),
            pltpu.VMEM((n_pad // NS,), jnp.float32),
            pltpu.VMEM_SHARED((NS, n_pad), jnp.float32),
            pltpu.VMEM_SHARED((NS, n_pad), jnp.float32),
        ),
    )
    def kern(src_hbm, tgt_hbm, p_hbm, q_hbm, den_hbm, cnt_hbm, ex_hbm,
             p_v, q_v, s_v, t_v, ex_v, den_v, cnt_v, acc_v, tmp_v,
             sh_den, sh_cnt):
        cid = lax.axis_index("c")
        sid = lax.axis_index("s")
        wid = sid * NC + cid
        base = wid * chunk
        pltpu.sync_copy(src_hbm.at[pl.ds(base, chunk)], s_v)
        pltpu.sync_copy(tgt_hbm.at[pl.ds(base, chunk)], t_v)
        pltpu.sync_copy(p_hbm, p_v)
        pltpu.sync_copy(q_hbm, q_v)

        zero16 = jnp.zeros((L,), jnp.float32)

        def zbody(i, c):
            den_v[pl.ds(i * L, L)] = zero16
            cnt_v[pl.ds(i * L, L)] = zero16
            return c

        lax.fori_loop(0, n_pad // L, zbody, 0)

        iota = lax.iota(jnp.int32, L)

        def ebody(i, c):
            sl = pl.ds(i * L, L)
            s16 = s_v[sl]
            t16 = t_v[sl]
            pv = plsc.load_gather(p_v, [s16])
            qv = plsc.load_gather(q_v, [t16])
            e = pv + qv
            e = jnp.where(e > 0.0, e, ALPHA * e)
            ex = jnp.exp(e)
            valid = (base + i * L + iota) < e_true
            ex = jnp.where(valid, ex, 0.0)
            ex_v[sl] = ex
            plsc.addupdate_scatter(den_v, [s16], ex)
            plsc.addupdate_scatter(cnt_v, [s16],
                                   jnp.where(valid, 1.0, 0.0))
            return c

        lax.fori_loop(0, nvec, ebody, 0)

        pltpu.sync_copy(ex_v, ex_hbm.at[pl.ds(base, chunk)])
        pltpu.sync_copy(den_v, sh_den.at[sid])
        pltpu.sync_copy(cnt_v, sh_cnt.at[sid])
        plsc.subcore_barrier()

        off = sid * nslice
        for sh, out in ((sh_den, den_hbm), (sh_cnt, cnt_hbm)):
            def z2(i, c):
                acc_v[pl.ds(i * L, L)] = zero16
                return c

            lax.fori_loop(0, nslice // L, z2, 0)
            for k in range(NS):
                pltpu.sync_copy(sh.at[k, pl.ds(off, nslice)], tmp_v)

                def abody(i, c):
                    sl = pl.ds(i * L, L)
                    acc_v[sl] = acc_v[sl] + tmp_v[sl]
                    return c

                lax.fori_loop(0, nslice // L, abody, 0)
            pltpu.sync_copy(acc_v, out.at[cid, pl.ds(off, nslice)])

    return kern(src_p, tgt_p, p_pad, q_pad)


def _tc_recip(den2, cnt2, n_true):
    n_pad = den2.shape[1]

    def body(d_ref, c_ref, r_ref):
        d = d_ref[...]
        c = c_ref[...]
        r_ref[...] = 1.0 / (d[0:1, :] + d[1:2, :] - c[0:1, :] - c[1:2, :]
                            + jnp.float32(n_true))

    return pl.pallas_call(
        body,
        out_shape=jax.ShapeDtypeStruct((1, n_pad), jnp.float32),
    )(den2, cnt2)


def _sc_pass2(src2d, tgt2d, ex_e, h, *, n_pad):
    epad = ex_e.shape[0]
    chunk = epad // NW
    nb = chunk // BE
    f = h.shape[1]
    nslice = n_pad // NS
    mesh = plsc.VectorSubcoreMesh(core_axis_name="c", subcore_axis_name="s")

    @functools.partial(
        pl.kernel,
        out_type=jax.ShapeDtypeStruct((NC, n_pad, f), jnp.float32),
        mesh=mesh,
        compiler_params=pltpu.CompilerParams(needs_layout_passes=False),
        scratch_types=(
            pltpu.VMEM((chunk,), jnp.float32),
            pltpu.VMEM((chunk // BE, BE), jnp.int32),
            pltpu.VMEM((chunk // BE, BE), jnp.int32),
            pltpu.VMEM((BE, 128), jnp.float32),
            pltpu.VMEM((BE, 128), jnp.float32),
            pltpu.VMEM_SHARED((n_pad, 128), jnp.float32),
            pltpu.SemaphoreType.DMA,
            pltpu.SemaphoreType.DMA,
        ),
    )
    def kern(s2_hbm, t2_hbm, ex_hbm, h_hbm, hp_hbm,
             ex_v, sidx2, tidx2, rows_a, rows_b, hp_sh, gsem_a, gsem_b):
        cid = lax.axis_index("c")
        sid = lax.axis_index("s")
        wid = sid * NC + cid
        base = wid * chunk
        brow = wid * nb
        pltpu.sync_copy(ex_hbm.at[pl.ds(base, chunk)], ex_v)
        pltpu.sync_copy(s2_hbm.at[pl.ds(brow, nb)], sidx2)
        pltpu.sync_copy(t2_hbm.at[pl.ds(brow, nb)], tidx2)

        def zrow(r, c):
            for fi in range(f // L):
                rows_a[r, pl.ds(fi * L, L)] = jnp.zeros((L,), jnp.float32)
            return c

        lax.fori_loop(0, BE, zrow, 0)
        off = sid * nslice
        for j in range(nslice // BE):
            pltpu.sync_copy(rows_a, hp_sh.at[pl.ds(off + j * BE, BE)])
        plsc.subcore_barrier()

        pltpu.async_copy(h_hbm.at[tidx2.at[0]], rows_a, gsem_a)
        pltpu.async_copy(h_hbm.at[tidx2.at[1]], rows_b, gsem_b)

        def bbody(j, c):
            for ib, rows, gsem in ((0, rows_a, gsem_a), (1, rows_b, gsem_b)):
                bi = 2 * j + ib
                pltpu.make_async_copy(h_hbm.at[tidx2.at[bi]], rows, gsem
                                      ).wait()

                def sgroup(g, c2):
                    ex16 = ex_v[pl.ds(bi * BE + g * L, L)]
                    for r2 in range(L):
                        bc = _bcast_lane(ex16, r2)
                        r = g * L + r2
                        for fi in range(f // L):
                            sl = pl.ds(fi * L, L)
                            rows[r, sl] = rows[r, sl] * bc
                    return c2

                pltpu.sync_copy(rows, hp_sh.at[sidx2.at[bi]], add=True)

                @pl.when(bi + 2 < nb)
                def _():
                    pltpu.async_copy(h_hbm.at[tidx2.at[bi + 2]], rows, gsem)
            return c

        lax.fori_loop(0, nb // 2, bbody, 0)
        plsc.subcore_barrier()
        pltpu.sync_copy(hp_sh.at[pl.ds(off, nslice)],
                        hp_hbm.at[cid, pl.ds(off, nslice)])

    return kern(src2d, tgt2d, ex_e, h)


def kernel(x, edge_index, W, a, Wp, bp):
    n, f = x.shape
    e_true = edge_index.shape[1]
    W0 = W[0]
    avec = a[0, :, 0]
    a2d = jnp.stack([avec[:f], avec[f:]], axis=1)

    h, pq = _tc_pre(x, W0, a2d)

    n_pad = -(-n // (NS * BE)) * (NS * BE)
    epad = -(-e_true // (NW * BE)) * (NW * BE)
    p_pad = jnp.pad(pq[:, 0], (0, n_pad - n))
    q_pad = jnp.pad(pq[:, 1], (0, n_pad - n))
    src_p = jnp.pad(edge_index[0], (0, epad - e_true))
    tgt_p = jnp.pad(edge_index[1], (0, epad - e_true))
    src2d = src_p.reshape(-1, BE)
    tgt2d = tgt_p.reshape(-1, BE)

    den2, cnt2, ex_e = _sc_pass1(src_p, tgt_p, p_pad, q_pad,
                                 n_pad=n_pad, n_true=n, e_true=e_true)
    rec_n = _tc_recip(den2, cnt2, n)[0, :n].reshape(n, 1)
    hp2 = _sc_pass2(src2d, tgt2d, ex_e, h, n_pad=n_pad)
    return _tc_post(hp2[0, :n], hp2[1, :n], rec_n, Wp, bp)


# speedup vs baseline: 1.3335x; 1.3335x over previous
import functools

import numpy as np

import jax
import jax.numpy as jnp
from jax import lax
from jax.experimental import pallas as pl
from jax.experimental.pallas import tpu as pltpu
from jax.experimental.pallas import tpu_sc as plsc

NC, NS, L = 2, 16, 16
NW = NC * NS
ALPHA = 0.2
BE = 128

_GATHER_1D = lax.GatherDimensionNumbers(
    offset_dims=(), collapsed_slice_dims=(0,), start_index_map=(0,))


def _bcast_lane(v16, lane):
    idx = jnp.full((L, 1), lane, jnp.int32)
    return lax.gather(v16, idx, _GATHER_1D, (1,),
                      mode=lax.GatherScatterMode.PROMISE_IN_BOUNDS)


def _tc_pre(x, W0, a2d):
    n, f = x.shape
    br = 1000

    def body(x_ref, w_ref, a_ref, h_ref, pq_ref):
        h = jnp.dot(x_ref[...], w_ref[...], preferred_element_type=jnp.float32)
        h_ref[...] = h.astype(jnp.bfloat16)
        pq_ref[...] = jnp.dot(h, a_ref[...], preferred_element_type=jnp.float32)

    return pl.pallas_call(
        body,
        grid=(n // br,),
        in_specs=[
            pl.BlockSpec((br, f), lambda i: (i, 0)),
            pl.BlockSpec((f, f), lambda i: (0, 0)),
            pl.BlockSpec((f, 2), lambda i: (0, 0)),
        ],
        out_specs=[
            pl.BlockSpec((br, f), lambda i: (i, 0)),
            pl.BlockSpec((br, 2), lambda i: (i, 0)),
        ],
        out_shape=[
            jax.ShapeDtypeStruct((n, f), jnp.bfloat16),
            jax.ShapeDtypeStruct((n, 2), jnp.float32),
        ],
    )(x, W0, a2d)


def _tc_post(hp0, hp1, rec_n, Wp, bp):
    n, f = hp0.shape
    br = 1000

    def body(h0_ref, h1_ref, r_ref, wp_ref, bp_ref, o_ref):
        hp = (h0_ref[...] + h1_ref[...]) * r_ref[...]
        o = lax.dot_general(hp, wp_ref[...], (((1,), (1,)), ((), ())),
                            preferred_element_type=jnp.float32)
        o_ref[...] = o + bp_ref[...]

    return pl.pallas_call(
        body,
        grid=(n // br,),
        in_specs=[
            pl.BlockSpec((br, f), lambda i: (i, 0)),
            pl.BlockSpec((br, f), lambda i: (i, 0)),
            pl.BlockSpec((br, 1), lambda i: (i, 0)),
            pl.BlockSpec((f, f), lambda i: (0, 0)),
            pl.BlockSpec((1, f), lambda i: (0, 0)),
        ],
        out_specs=pl.BlockSpec((br, f), lambda i: (i, 0)),
        out_shape=jax.ShapeDtypeStruct((n, f), jnp.float32),
    )(hp0, hp1, rec_n, Wp, bp.reshape(1, f))


def _sc_pass1(src_p, tgt_p, p_pad, q_pad, *, n_pad, n_true, e_true):
    epad = src_p.shape[0]
    chunk = epad // NW
    nvec = chunk // L
    nslice = n_pad // NS
    mesh = plsc.VectorSubcoreMesh(core_axis_name="c", subcore_axis_name="s")

    @functools.partial(
        pl.kernel,
        out_type=(
            jax.ShapeDtypeStruct((NC, n_pad), jnp.float32),
            jax.ShapeDtypeStruct((NC, n_pad), jnp.float32),
            jax.ShapeDtypeStruct((epad,), jnp.float32),
        ),
        mesh=mesh,
        compiler_params=pltpu.CompilerParams(needs_layout_passes=False),
        scratch_types=(
            pltpu.VMEM((n_pad,), jnp.float32),
            pltpu.VMEM((n_pad,), jnp.float32),
            pltpu.VMEM((chunk,), jnp.int32),
            pltpu.VMEM((chunk,), jnp.int32),
            pltpu.VMEM((chunk,), jnp.float32),
            pltpu.VMEM((n_pad,), jnp.float32),
            pltpu.VMEM((n_pad,), jnp.float32),
            pltpu.VMEM((n_pad // NS,), jnp.float32),
            pltpu.VMEM((n_pad // NS,), jnp.float32),
            pltpu.VMEM_SHARED((NS, n_pad), jnp.float32),
            pltpu.VMEM_SHARED((NS, n_pad), jnp.float32),
        ),
    )
    def kern(src_hbm, tgt_hbm, p_hbm, q_hbm, den_hbm, cnt_hbm, ex_hbm,
             p_v, q_v, s_v, t_v, ex_v, den_v, cnt_v, acc_v, tmp_v,
             sh_den, sh_cnt):
        cid = lax.axis_index("c")
        sid = lax.axis_index("s")
        wid = sid * NC + cid
        base = wid * chunk
        pltpu.sync_copy(src_hbm.at[pl.ds(base, chunk)], s_v)
        pltpu.sync_copy(tgt_hbm.at[pl.ds(base, chunk)], t_v)
        pltpu.sync_copy(p_hbm, p_v)
        pltpu.sync_copy(q_hbm, q_v)

        zero16 = jnp.zeros((L,), jnp.float32)

        def zbody(i, c):
            den_v[pl.ds(i * L, L)] = zero16
            cnt_v[pl.ds(i * L, L)] = zero16
            return c

        lax.fori_loop(0, n_pad // L, zbody, 0)

        iota = lax.iota(jnp.int32, L)

        def ebody(i, c):
            sl = pl.ds(i * L, L)
            s16 = s_v[sl]
            t16 = t_v[sl]
            pv = plsc.load_gather(p_v, [s16])
            qv = plsc.load_gather(q_v, [t16])
            e = pv + qv
            e = jnp.where(e > 0.0, e, ALPHA * e)
            ex = jnp.exp(e)
            valid = (base + i * L + iota) < e_true
            ex = jnp.where(valid, ex, 0.0)
            ex_v[sl] = ex
            plsc.addupdate_scatter(den_v, [s16], ex)
            plsc.addupdate_scatter(cnt_v, [s16],
                                   jnp.where(valid, 1.0, 0.0))
            return c

        lax.fori_loop(0, nvec, ebody, 0)

        pltpu.sync_copy(ex_v, ex_hbm.at[pl.ds(base, chunk)])
        pltpu.sync_copy(den_v, sh_den.at[sid])
        pltpu.sync_copy(cnt_v, sh_cnt.at[sid])
        plsc.subcore_barrier()

        off = sid * nslice
        for sh, out in ((sh_den, den_hbm), (sh_cnt, cnt_hbm)):
            def z2(i, c):
                acc_v[pl.ds(i * L, L)] = zero16
                return c

            lax.fori_loop(0, nslice // L, z2, 0)
            for k in range(NS):
                pltpu.sync_copy(sh.at[k, pl.ds(off, nslice)], tmp_v)

                def abody(i, c):
                    sl = pl.ds(i * L, L)
                    acc_v[sl] = acc_v[sl] + tmp_v[sl]
                    return c

                lax.fori_loop(0, nslice // L, abody, 0)
            pltpu.sync_copy(acc_v, out.at[cid, pl.ds(off, nslice)])

    return kern(src_p, tgt_p, p_pad, q_pad)


def _tc_recip(den2, cnt2, n_true):
    n_pad = den2.shape[1]

    def body(d_ref, c_ref, r_ref):
        d = d_ref[...]
        c = c_ref[...]
        r_ref[...] = 1.0 / (d[0:1, :] + d[1:2, :] - c[0:1, :] - c[1:2, :]
                            + jnp.float32(n_true))

    return pl.pallas_call(
        body,
        out_shape=jax.ShapeDtypeStruct((1, n_pad), jnp.float32),
    )(den2, cnt2)


def _sc_pass2(src2d, tgt2d, ex_e, h64, *, n_pad):
    epad = ex_e.shape[0]
    chunk = epad // NW
    be = 64
    nbuf = 2
    nb = chunk // be
    f = 2 * h64.shape[1]
    nslice = n_pad // NS
    mesh = plsc.VectorSubcoreMesh(core_axis_name="c", subcore_axis_name="s")

    @functools.partial(
        pl.kernel,
        out_type=jax.ShapeDtypeStruct((NC, n_pad, f), jnp.float32),
        mesh=mesh,
        compiler_params=pltpu.CompilerParams(needs_layout_passes=False,
                                             use_tc_tiling_on_sc=False),
        scratch_types=(
            pltpu.VMEM((chunk,), jnp.float32),
            pltpu.VMEM((chunk // be, be), jnp.int32),
            pltpu.VMEM((chunk // be, be), jnp.int32),
            pltpu.VMEM((be, 64), jnp.int32),
            pltpu.VMEM((be, 64), jnp.int32),
            pltpu.VMEM((be, 128), jnp.float32),
            pltpu.VMEM((be, 128), jnp.float32),
            pltpu.VMEM_SHARED((n_pad, 128), jnp.float32),
            pltpu.SemaphoreType.DMA,
            pltpu.SemaphoreType.DMA,
        ),
    )
    def kern(s2_hbm, t2_hbm, ex_hbm, h_hbm, hp_hbm,
             ex_v, sidx2, tidx2, gb_0, gb_1, ob_0, ob_1, hp_sh,
             gsem_0, gsem_1):
        cid = lax.axis_index("c")
        sid = lax.axis_index("s")
        wid = sid * NC + cid
        base = wid * chunk
        brow = wid * nb
        pltpu.sync_copy(ex_hbm.at[pl.ds(base, chunk)], ex_v)
        pltpu.sync_copy(s2_hbm.at[pl.ds(brow, nb)], sidx2)
        pltpu.sync_copy(t2_hbm.at[pl.ds(brow, nb)], tidx2)

        def zrow(r, c):
            for fi in range(f // L):
                ob_0[r, pl.ds(fi * L, L)] = jnp.zeros((L,), jnp.float32)
            return c

        lax.fori_loop(0, be, zrow, 0)
        off = sid * nslice
        for j in range(nslice // be):
            pltpu.sync_copy(ob_0, hp_sh.at[pl.ds(off + j * be, be)])
        plsc.subcore_barrier()

        bufs = ((gb_0, ob_0, gsem_0), (gb_1, ob_1, gsem_1))
        for ib, (gb, ob, gsem) in enumerate(bufs):
            pltpu.async_copy(h_hbm.at[tidx2.at[ib]], gb, gsem)

        def bbody(j, c):
            for ib, (gb, ob, gsem) in enumerate(bufs):
                bi = nbuf * j + ib
                pltpu.make_async_copy(h_hbm.at[tidx2.at[bi]], gb, gsem
                                      ).wait()

                def sgroup(g, c2):
                    ex16 = ex_v[pl.ds(bi * be + g * L, L)]
                    for r2 in range(L):
                        bc = _bcast_lane(ex16, r2)
                        r = g * L + r2
                        for fi in range(f // 32):
                            v16 = gb[r, pl.ds(fi * L, L)]
                            v32 = plsc.bitcast(v16, jnp.bfloat16)
                            ev, ov = plsc.unpack(
                                v32, format=plsc.PackFormat.INTERLEAVED)
                            ob[r, pl.ds(fi * 32, L)] = ev * bc
                            ob[r, pl.ds(fi * 32 + L, L)] = ov * bc
                    return c2

                lax.fori_loop(0, be // L, sgroup, 0)

                @pl.when(bi + nbuf < nb)
                def _():
                    pltpu.async_copy(h_hbm.at[tidx2.at[bi + nbuf]], gb, gsem)

                pltpu.sync_copy(ob, hp_sh.at[sidx2.at[bi]], add=True)
            return c

        lax.fori_loop(0, nb // nbuf, bbody, 0)
        plsc.subcore_barrier()
        pltpu.sync_copy(hp_sh.at[pl.ds(off, nslice)],
                        hp_hbm.at[cid, pl.ds(off, nslice)])

    return kern(src2d, tgt2d, ex_e, h64)


def kernel(x, edge_index, W, a, Wp, bp):
    n, f = x.shape
    e_true = edge_index.shape[1]
    W0 = W[0]
    avec = a[0, :, 0]
    a2d = jnp.stack([avec[:f], avec[f:]], axis=1)

    h, pq = _tc_pre(x, W0, a2d)

    n_pad = -(-n // (NS * BE)) * (NS * BE)
    epad = -(-e_true // (NW * BE)) * (NW * BE)
    p_pad = jnp.pad(pq[:, 0], (0, n_pad - n))
    q_pad = jnp.pad(pq[:, 1], (0, n_pad - n))
    src_p = jnp.pad(edge_index[0], (0, epad - e_true))
    tgt_p = jnp.pad(edge_index[1], (0, epad - e_true))
    src2d = src_p.reshape(-1, 64)
    tgt2d = tgt_p.reshape(-1, 64)

    den2, cnt2, ex_e = _sc_pass1(src_p, tgt_p, p_pad, q_pad,
                                 n_pad=n_pad, n_true=n, e_true=e_true)
    rec_n = _tc_recip(den2, cnt2, n)[0, :n].reshape(n, 1)
    h64 = lax.bitcast_convert_type(h.reshape(n, f // 2, 2), jnp.int32)
    hp2 = _sc_pass2(src2d, tgt2d, ex_e, h64, n_pad=n_pad)
    blk = np.arange(0, f, 32)[:, None]
    half = np.concatenate([2 * np.arange(16), 2 * np.arange(16) + 1])[None, :]
    idx_list = jnp.asarray((blk + half).reshape(-1), dtype=jnp.int32)
    Wp_perm = Wp[:, idx_list]
    return _tc_post(hp2[0, :n], hp2[1, :n], rec_n, Wp_perm, bp)


# speedup vs baseline: 1.3585x; 1.0188x over previous
import functools

import numpy as np

import jax
import jax.numpy as jnp
from jax import lax
from jax.experimental import pallas as pl
from jax.experimental.pallas import tpu as pltpu
from jax.experimental.pallas import tpu_sc as plsc

NC, NS, L = 2, 16, 16
NW = NC * NS
ALPHA = 0.2
BE = 128

_GATHER_1D = lax.GatherDimensionNumbers(
    offset_dims=(), collapsed_slice_dims=(0,), start_index_map=(0,))


def _bcast_lane(v16, lane):
    idx = jnp.full((L, 1), lane, jnp.int32)
    return lax.gather(v16, idx, _GATHER_1D, (1,),
                      mode=lax.GatherScatterMode.PROMISE_IN_BOUNDS)


def _tc_pre(x, W0, a2d):
    n, f = x.shape
    br = 1000

    def body(x_ref, w_ref, a_ref, h_ref, pq_ref):
        h = jnp.dot(x_ref[...], w_ref[...], preferred_element_type=jnp.float32)
        h_ref[...] = h.astype(jnp.bfloat16)
        pq_ref[...] = jnp.dot(h, a_ref[...], preferred_element_type=jnp.float32)

    return pl.pallas_call(
        body,
        grid=(n // br,),
        in_specs=[
            pl.BlockSpec((br, f), lambda i: (i, 0)),
            pl.BlockSpec((f, f), lambda i: (0, 0)),
            pl.BlockSpec((f, 2), lambda i: (0, 0)),
        ],
        out_specs=[
            pl.BlockSpec((br, f), lambda i: (i, 0)),
            pl.BlockSpec((br, 2), lambda i: (i, 0)),
        ],
        out_shape=[
            jax.ShapeDtypeStruct((n, f), jnp.bfloat16),
            jax.ShapeDtypeStruct((n, 2), jnp.float32),
        ],
    )(x, W0, a2d)


def _tc_post(hp0, hp1, rec_n, Wp, bp):
    n, f = hp0.shape
    br = 1000

    def body(h0_ref, h1_ref, r_ref, wp_ref, bp_ref, o_ref):
        hp = (h0_ref[...] + h1_ref[...]) * r_ref[...]
        o = lax.dot_general(hp, wp_ref[...], (((1,), (1,)), ((), ())),
                            preferred_element_type=jnp.float32)
        o_ref[...] = o + bp_ref[...]

    return pl.pallas_call(
        body,
        grid=(n // br,),
        in_specs=[
            pl.BlockSpec((br, f), lambda i: (i, 0)),
            pl.BlockSpec((br, f), lambda i: (i, 0)),
            pl.BlockSpec((br, 1), lambda i: (i, 0)),
            pl.BlockSpec((f, f), lambda i: (0, 0)),
            pl.BlockSpec((1, f), lambda i: (0, 0)),
        ],
        out_specs=pl.BlockSpec((br, f), lambda i: (i, 0)),
        out_shape=jax.ShapeDtypeStruct((n, f), jnp.float32),
    )(hp0, hp1, rec_n, Wp, bp.reshape(1, f))


def _sc_pass1(src_p, tgt_p, p_pad, q_pad, *, n_pad, n_true, e_true):
    epad = src_p.shape[0]
    chunk = epad // NW
    nvec = chunk // L
    nslice = n_pad // NS
    mesh = plsc.VectorSubcoreMesh(core_axis_name="c", subcore_axis_name="s")

    @functools.partial(
        pl.kernel,
        out_type=(
            jax.ShapeDtypeStruct((NC, n_pad), jnp.float32),
            jax.ShapeDtypeStruct((epad,), jnp.float32),
        ),
        mesh=mesh,
        compiler_params=pltpu.CompilerParams(needs_layout_passes=False),
        scratch_types=(
            pltpu.VMEM((n_pad,), jnp.float32),
            pltpu.VMEM((n_pad,), jnp.float32),
            pltpu.VMEM((chunk,), jnp.int32),
            pltpu.VMEM((chunk,), jnp.int32),
            pltpu.VMEM((chunk,), jnp.float32),
            pltpu.VMEM((n_pad,), jnp.float32),
            pltpu.VMEM((n_pad // NS,), jnp.float32),
            pltpu.VMEM((n_pad // NS,), jnp.float32),
            pltpu.VMEM_SHARED((NS, n_pad), jnp.float32),
        ),
    )
    def kern(src_hbm, tgt_hbm, p_hbm, q_hbm, den_hbm, ex_hbm,
             p_v, q_v, s_v, t_v, ex_v, den_v, acc_v, tmp_v, sh_den):
        cid = lax.axis_index("c")
        sid = lax.axis_index("s")
        wid = sid * NC + cid
        base = wid * chunk
        pltpu.sync_copy(src_hbm.at[pl.ds(base, chunk)], s_v)
        pltpu.sync_copy(tgt_hbm.at[pl.ds(base, chunk)], t_v)
        pltpu.sync_copy(p_hbm, p_v)
        pltpu.sync_copy(q_hbm, q_v)

        zero16 = jnp.zeros((L,), jnp.float32)

        def zbody(i, c):
            den_v[pl.ds(i * L, L)] = zero16
            return c

        lax.fori_loop(0, n_pad // L, zbody, 0)

        iota = lax.iota(jnp.int32, L)

        def ebody(i, c):
            sl = pl.ds(i * L, L)
            s16 = s_v[sl]
            t16 = t_v[sl]
            pv = plsc.load_gather(p_v, [s16])
            qv = plsc.load_gather(q_v, [t16])
            e = pv + qv
            e = jnp.where(e > 0.0, e, ALPHA * e)
            ex = jnp.exp(e)
            valid = (base + i * L + iota) < e_true
            ex_v[sl] = jnp.where(valid, ex, 0.0)
            plsc.addupdate_scatter(den_v, [s16],
                                   jnp.where(valid, ex - 1.0, 0.0))
            return c

        lax.fori_loop(0, nvec, ebody, 0)

        pltpu.sync_copy(ex_v, ex_hbm.at[pl.ds(base, chunk)])
        pltpu.sync_copy(den_v, sh_den.at[sid])
        plsc.subcore_barrier()

        off = sid * nslice
        for sh, out in ((sh_den, den_hbm),):
            def z2(i, c):
                acc_v[pl.ds(i * L, L)] = zero16
                return c

            lax.fori_loop(0, nslice // L, z2, 0)
            for k in range(NS):
                pltpu.sync_copy(sh.at[k, pl.ds(off, nslice)], tmp_v)

                def abody(i, c):
                    sl = pl.ds(i * L, L)
                    acc_v[sl] = acc_v[sl] + tmp_v[sl]
                    return c

                lax.fori_loop(0, nslice // L, abody, 0)
            pltpu.sync_copy(acc_v, out.at[cid, pl.ds(off, nslice)])

    return kern(src_p, tgt_p, p_pad, q_pad)


def _tc_recip(den2, n_true):
    n_pad = den2.shape[1]

    def body(d_ref, r_ref):
        d = d_ref[...]
        r_ref[...] = 1.0 / (d[0:1, :] + d[1:2, :] + jnp.float32(n_true))

    return pl.pallas_call(
        body,
        out_shape=jax.ShapeDtypeStruct((1, n_pad), jnp.float32),
    )(den2)


def _sc_pass2(src2d, tgt2d, ex_e, h64, *, n_pad):
    epad = ex_e.shape[0]
    chunk = epad // NW
    be = 64
    nbuf = 4
    nb = chunk // be
    f = 2 * h64.shape[1]
    nslice = n_pad // NS
    mesh = plsc.VectorSubcoreMesh(core_axis_name="c", subcore_axis_name="s")

    @functools.partial(
        pl.kernel,
        out_type=jax.ShapeDtypeStruct((NC, n_pad, f), jnp.float32),
        mesh=mesh,
        compiler_params=pltpu.CompilerParams(needs_layout_passes=False,
                                             use_tc_tiling_on_sc=False),
        scratch_types=(
            pltpu.VMEM((chunk,), jnp.float32),
            pltpu.VMEM((chunk // be, be), jnp.int32),
            pltpu.VMEM((chunk // be, be), jnp.int32),
            pltpu.VMEM((be, 64), jnp.int32),
            pltpu.VMEM((be, 64), jnp.int32),
            pltpu.VMEM((be, 64), jnp.int32),
            pltpu.VMEM((be, 64), jnp.int32),
            pltpu.VMEM((be, 128), jnp.float32),
            pltpu.VMEM((be, 128), jnp.float32),
            pltpu.VMEM_SHARED((n_pad, 128), jnp.float32),
            pltpu.SemaphoreType.DMA,
            pltpu.SemaphoreType.DMA,
            pltpu.SemaphoreType.DMA,
            pltpu.SemaphoreType.DMA,
        ),
    )
    def kern(s2_hbm, t2_hbm, ex_hbm, h_hbm, hp_hbm,
             ex_v, sidx2, tidx2, gb_0, gb_1, gb_2, gb_3, ob_0, ob_1, hp_sh,
             gsem_0, gsem_1, gsem_2, gsem_3):
        cid = lax.axis_index("c")
        sid = lax.axis_index("s")
        wid = sid * NC + cid
        base = wid * chunk
        brow = wid * nb
        pltpu.sync_copy(ex_hbm.at[pl.ds(base, chunk)], ex_v)
        pltpu.sync_copy(s2_hbm.at[pl.ds(brow, nb)], sidx2)
        pltpu.sync_copy(t2_hbm.at[pl.ds(brow, nb)], tidx2)

        def zrow(r, c):
            for fi in range(f // L):
                ob_0[r, pl.ds(fi * L, L)] = jnp.zeros((L,), jnp.float32)
            return c

        lax.fori_loop(0, be, zrow, 0)
        off = sid * nslice
        for j in range(nslice // be):
            pltpu.sync_copy(ob_0, hp_sh.at[pl.ds(off + j * be, be)])
        plsc.subcore_barrier()

        bufs = ((gb_0, ob_0, gsem_0), (gb_1, ob_1, gsem_1),
                (gb_2, ob_0, gsem_2), (gb_3, ob_1, gsem_3))
        for ib, (gb, ob, gsem) in enumerate(bufs):
            pltpu.async_copy(h_hbm.at[tidx2.at[ib]], gb, gsem)

        def bbody(j, c):
            for ib, (gb, ob, gsem) in enumerate(bufs):
                bi = nbuf * j + ib
                pltpu.make_async_copy(h_hbm.at[tidx2.at[bi]], gb, gsem
                                      ).wait()

                def sgroup(g, c2):
                    ex16 = ex_v[pl.ds(bi * be + g * L, L)]
                    for r2 in range(L):
                        bc = _bcast_lane(ex16, r2)
                        r = g * L + r2
                        for fi in range(f // 32):
                            v16 = gb[r, pl.ds(fi * L, L)]
                            v32 = plsc.bitcast(v16, jnp.bfloat16)
                            ev, ov = plsc.unpack(
                                v32, format=plsc.PackFormat.INTERLEAVED)
                            ob[r, pl.ds(fi * 32, L)] = ev * bc
                            ob[r, pl.ds(fi * 32 + L, L)] = ov * bc
                    return c2

                lax.fori_loop(0, be // L, sgroup, 0)

                @pl.when(bi + nbuf < nb)
                def _():
                    pltpu.async_copy(h_hbm.at[tidx2.at[bi + nbuf]], gb, gsem)

                pltpu.sync_copy(ob, hp_sh.at[sidx2.at[bi]], add=True)
            return c

        lax.fori_loop(0, nb // nbuf, bbody, 0)
        plsc.subcore_barrier()
        pltpu.sync_copy(hp_sh.at[pl.ds(off, nslice)],
                        hp_hbm.at[cid, pl.ds(off, nslice)])

    return kern(src2d, tgt2d, ex_e, h64)


def kernel(x, edge_index, W, a, Wp, bp):
    n, f = x.shape
    e_true = edge_index.shape[1]
    W0 = W[0]
    avec = a[0, :, 0]
    a2d = jnp.stack([avec[:f], avec[f:]], axis=1)

    h, pq = _tc_pre(x, W0, a2d)

    n_pad = -(-n // (NS * BE)) * (NS * BE)
    epad = -(-e_true // (NW * BE)) * (NW * BE)
    p_pad = jnp.pad(pq[:, 0], (0, n_pad - n))
    q_pad = jnp.pad(pq[:, 1], (0, n_pad - n))
    src_p = jnp.pad(edge_index[0], (0, epad - e_true))
    tgt_p = jnp.pad(edge_index[1], (0, epad - e_true))
    src2d = src_p.reshape(-1, 64)
    tgt2d = tgt_p.reshape(-1, 64)

    den2, ex_e = _sc_pass1(src_p, tgt_p, p_pad, q_pad,
                           n_pad=n_pad, n_true=n, e_true=e_true)
    rec_n = _tc_recip(den2, n)[0, :n].reshape(n, 1)
    h64 = lax.bitcast_convert_type(h.reshape(n, f // 2, 2), jnp.int32)
    hp2 = _sc_pass2(src2d, tgt2d, ex_e, h64, n_pad=n_pad)
    blk = np.arange(0, f, 32)[:, None]
    half = np.concatenate([2 * np.arange(16), 2 * np.arange(16) + 1])[None, :]
    idx_list = jnp.asarray((blk + half).reshape(-1), dtype=jnp.int32)
    Wp_perm = Wp[:, idx_list]
    return _tc_post(hp2[0, :n], hp2[1, :n], rec_n, Wp_perm, bp)


# speedup vs baseline: 1.3632x; 1.0034x over previous
import functools

import numpy as np

import jax
import jax.numpy as jnp
from jax import lax
from jax.experimental import pallas as pl
from jax.experimental.pallas import tpu as pltpu
from jax.experimental.pallas import tpu_sc as plsc

NC, NS, L = 2, 16, 16
NW = NC * NS
ALPHA = 0.2
BE = 128

_GATHER_1D = lax.GatherDimensionNumbers(
    offset_dims=(), collapsed_slice_dims=(0,), start_index_map=(0,))


def _bcast_lane(v16, lane):
    idx = jnp.full((L, 1), lane, jnp.int32)
    return lax.gather(v16, idx, _GATHER_1D, (1,),
                      mode=lax.GatherScatterMode.PROMISE_IN_BOUNDS)


def _tc_pre(x, W0, a2d):
    n, f = x.shape
    br = 1000

    def body(x_ref, w_ref, a_ref, h_ref, pq_ref):
        h = jnp.dot(x_ref[...], w_ref[...], preferred_element_type=jnp.float32)
        h_ref[...] = h.astype(jnp.bfloat16)
        pq_ref[...] = jnp.dot(h, a_ref[...], preferred_element_type=jnp.float32)

    return pl.pallas_call(
        body,
        grid=(n // br,),
        in_specs=[
            pl.BlockSpec((br, f), lambda i: (i, 0)),
            pl.BlockSpec((f, f), lambda i: (0, 0)),
            pl.BlockSpec((f, 2), lambda i: (0, 0)),
        ],
        out_specs=[
            pl.BlockSpec((br, f), lambda i: (i, 0)),
            pl.BlockSpec((br, 2), lambda i: (i, 0)),
        ],
        out_shape=[
            jax.ShapeDtypeStruct((n, f), jnp.bfloat16),
            jax.ShapeDtypeStruct((n, 2), jnp.float32),
        ],
    )(x, W0, a2d)


def _tc_post(hp0, hp1, d0n, d1n, n_true, Wp, bp):
    n, f = hp0.shape
    br = 1000

    def body(h0_ref, h1_ref, d0_ref, d1_ref, wp_ref, bp_ref, o_ref):
        rec = 1.0 / (d0_ref[...] + d1_ref[...] + jnp.float32(n_true))
        hp = (h0_ref[...] + h1_ref[...]) * rec
        o = lax.dot_general(hp, wp_ref[...], (((1,), (1,)), ((), ())),
                            preferred_element_type=jnp.float32)
        o_ref[...] = o + bp_ref[...]

    return pl.pallas_call(
        body,
        grid=(n // br,),
        in_specs=[
            pl.BlockSpec((br, f), lambda i: (i, 0)),
            pl.BlockSpec((br, f), lambda i: (i, 0)),
            pl.BlockSpec((br, 1), lambda i: (i, 0)),
            pl.BlockSpec((br, 1), lambda i: (i, 0)),
            pl.BlockSpec((f, f), lambda i: (0, 0)),
            pl.BlockSpec((1, f), lambda i: (0, 0)),
        ],
        out_specs=pl.BlockSpec((br, f), lambda i: (i, 0)),
        out_shape=jax.ShapeDtypeStruct((n, f), jnp.float32),
    )(hp0, hp1, d0n, d1n, Wp, bp.reshape(1, f))


def _sc_pass1(src_p, tgt_p, p_pad, q_pad, *, n_pad, n_true, e_true):
    epad = src_p.shape[0]
    chunk = epad // NW
    nvec = chunk // L
    nslice = n_pad // NS
    mesh = plsc.VectorSubcoreMesh(core_axis_name="c", subcore_axis_name="s")

    @functools.partial(
        pl.kernel,
        out_type=(
            jax.ShapeDtypeStruct((n_pad,), jnp.float32),
            jax.ShapeDtypeStruct((n_pad,), jnp.float32),
            jax.ShapeDtypeStruct((epad,), jnp.float32),
        ),
        mesh=mesh,
        compiler_params=pltpu.CompilerParams(needs_layout_passes=False),
        scratch_types=(
            pltpu.VMEM((n_pad,), jnp.float32),
            pltpu.VMEM((n_pad,), jnp.float32),
            pltpu.VMEM((chunk,), jnp.int32),
            pltpu.VMEM((chunk,), jnp.int32),
            pltpu.VMEM((chunk,), jnp.float32),
            pltpu.VMEM((n_pad,), jnp.float32),
            pltpu.VMEM((n_pad // NS,), jnp.float32),
            pltpu.VMEM((n_pad // NS,), jnp.float32),
            pltpu.VMEM_SHARED((NS, n_pad), jnp.float32),
        ),
    )
    def kern(src_hbm, tgt_hbm, p_hbm, q_hbm, den0_hbm, den1_hbm, ex_hbm,
             p_v, q_v, s_v, t_v, ex_v, den_v, acc_v, tmp_v, sh_den):
        cid = lax.axis_index("c")
        sid = lax.axis_index("s")
        wid = sid * NC + cid
        base = wid * chunk
        pltpu.sync_copy(src_hbm.at[pl.ds(base, chunk)], s_v)
        pltpu.sync_copy(tgt_hbm.at[pl.ds(base, chunk)], t_v)
        pltpu.sync_copy(p_hbm, p_v)
        pltpu.sync_copy(q_hbm, q_v)

        zero16 = jnp.zeros((L,), jnp.float32)

        def zbody(i, c):
            den_v[pl.ds(i * L, L)] = zero16
            return c

        lax.fori_loop(0, n_pad // L, zbody, 0)

        iota = lax.iota(jnp.int32, L)

        def ebody(i, c):
            sl = pl.ds(i * L, L)
            s16 = s_v[sl]
            t16 = t_v[sl]
            pv = plsc.load_gather(p_v, [s16])
            qv = plsc.load_gather(q_v, [t16])
            e = pv + qv
            e = jnp.where(e > 0.0, e, ALPHA * e)
            ex = jnp.exp(e)
            valid = (base + i * L + iota) < e_true
            ex_v[sl] = jnp.where(valid, ex, 0.0)
            plsc.addupdate_scatter(den_v, [s16],
                                   jnp.where(valid, ex - 1.0, 0.0))
            return c

        lax.fori_loop(0, nvec, ebody, 0)

        pltpu.sync_copy(ex_v, ex_hbm.at[pl.ds(base, chunk)])
        pltpu.sync_copy(den_v, sh_den.at[sid])
        plsc.subcore_barrier()

        off = sid * nslice

        def z2(i, c):
            acc_v[pl.ds(i * L, L)] = zero16
            return c

        lax.fori_loop(0, nslice // L, z2, 0)
        for k in range(NS):
            pltpu.sync_copy(sh_den.at[k, pl.ds(off, nslice)], tmp_v)

            def abody(i, c):
                sl = pl.ds(i * L, L)
                acc_v[sl] = acc_v[sl] + tmp_v[sl]
                return c

            lax.fori_loop(0, nslice // L, abody, 0)

        @pl.when(cid == 0)
        def _():
            pltpu.sync_copy(acc_v, den0_hbm.at[pl.ds(off, nslice)])

        @pl.when(cid == 1)
        def _():
            pltpu.sync_copy(acc_v, den1_hbm.at[pl.ds(off, nslice)])

    return kern(src_p, tgt_p, p_pad, q_pad)


def _sc_pass1(src_p, tgt_p, p_pad, q_pad, *, n_pad, n_true, e_true):
    epad = src_p.shape[0]
    chunk = epad // NW
    nvec = chunk // L
    nslice = n_pad // NS
    mesh = plsc.VectorSubcoreMesh(core_axis_name="c", subcore_axis_name="s")

    @functools.partial(
        pl.kernel,
        out_type=(
            jax.ShapeDtypeStruct((n_pad,), jnp.float32),
            jax.ShapeDtypeStruct((n_pad,), jnp.float32),
            jax.ShapeDtypeStruct((epad,), jnp.float32),
        ),
        mesh=mesh,
        compiler_params=pltpu.CompilerParams(needs_layout_passes=False),
        scratch_types=(
            pltpu.VMEM((n_pad,), jnp.float32),
            pltpu.VMEM((n_pad,), jnp.float32),
            pltpu.VMEM((chunk,), jnp.int32),
            pltpu.VMEM((chunk,), jnp.int32),
            pltpu.VMEM((chunk,), jnp.float32),
            pltpu.VMEM((n_pad,), jnp.float32),
            pltpu.VMEM((n_pad // NS,), jnp.float32),
            pltpu.VMEM((n_pad // NS,), jnp.float32),
            pltpu.VMEM_SHARED((NS, n_pad), jnp.float32),
        ),
    )
    def kern(src_hbm, tgt_hbm, p_hbm, q_hbm, den0_hbm, den1_hbm, ex_hbm,
             p_v, q_v, s_v, t_v, ex_v, den_v, acc_v, tmp_v, sh_den):
        cid = lax.axis_index("c")
        sid = lax.axis_index("s")
        wid = sid * NC + cid
        base = wid * chunk
        pltpu.sync_copy(src_hbm.at[pl.ds(base, chunk)], s_v)
        pltpu.sync_copy(tgt_hbm.at[pl.ds(base, chunk)], t_v)
        pltpu.sync_copy(p_hbm, p_v)
        pltpu.sync_copy(q_hbm, q_v)

        zero16 = jnp.zeros((L,), jnp.float32)

        def zbody(i, c):
            den_v[pl.ds(i * L, L)] = zero16
            return c

        lax.fori_loop(0, n_pad // L, zbody, 0)

        iota = lax.iota(jnp.int32, L)

        def ebody(i, c):
            sl = pl.ds(i * L, L)
            s16 = s_v[sl]
            t16 = t_v[sl]
            pv = plsc.load_gather(p_v, [s16])
            qv = plsc.load_gather(q_v, [t16])
            e = pv + qv
            e = jnp.where(e > 0.0, e, ALPHA * e)
            ex = jnp.exp(e)
            valid = (base + i * L + iota) < e_true
            ex_v[sl] = jnp.where(valid, ex, 0.0)
            plsc.addupdate_scatter(den_v, [s16],
                                   jnp.where(valid, ex - 1.0, 0.0))
            return c

        lax.fori_loop(0, nvec, ebody, 0)

        pltpu.sync_copy(ex_v, ex_hbm.at[pl.ds(base, chunk)])
        pltpu.sync_copy(den_v, sh_den.at[sid])
        plsc.subcore_barrier()

        off = sid * nslice

        def z2(i, c):
            acc_v[pl.ds(i * L, L)] = zero16
            return c

        lax.fori_loop(0, nslice // L, z2, 0)
        for k in range(NS):
            pltpu.sync_copy(sh_den.at[k, pl.ds(off, nslice)], tmp_v)

            def abody(i, c):
                sl = pl.ds(i * L, L)
                acc_v[sl] = acc_v[sl] + tmp_v[sl]
                return c

            lax.fori_loop(0, nslice // L, abody, 0)

        @pl.when(cid == 0)
        def _():
            pltpu.sync_copy(acc_v, den0_hbm.at[pl.ds(off, nslice)])

        @pl.when(cid == 1)
        def _():
            pltpu.sync_copy(acc_v, den1_hbm.at[pl.ds(off, nslice)])

    return kern(src_p, tgt_p, p_pad, q_pad)


def _tc_recip(den2, n_true):
    n_pad = den2.shape[1]

    def body(d_ref, r_ref):
        d = d_ref[...]
        r_ref[...] = 1.0 / (d[0:1, :] + d[1:2, :] + jnp.float32(n_true))

    return pl.pallas_call(
        body,
        out_shape=jax.ShapeDtypeStruct((1, n_pad), jnp.float32),
    )(den2)


def _sc_pass2(src2d, tgt2d, ex_e, h64, *, n_pad):
    epad = ex_e.shape[0]
    chunk = epad // NW
    be = 64
    nbuf = 4
    nb = chunk // be
    f = 2 * h64.shape[1]
    nslice = n_pad // NS
    mesh = plsc.VectorSubcoreMesh(core_axis_name="c", subcore_axis_name="s")

    @functools.partial(
        pl.kernel,
        out_type=jax.ShapeDtypeStruct((NC, n_pad, f), jnp.float32),
        mesh=mesh,
        compiler_params=pltpu.CompilerParams(needs_layout_passes=False,
                                             use_tc_tiling_on_sc=False),
        scratch_types=(
            pltpu.VMEM((chunk,), jnp.float32),
            pltpu.VMEM((chunk // be, be), jnp.int32),
            pltpu.VMEM((chunk // be, be), jnp.int32),
            pltpu.VMEM((be, 64), jnp.int32),
            pltpu.VMEM((be, 64), jnp.int32),
            pltpu.VMEM((be, 64), jnp.int32),
            pltpu.VMEM((be, 64), jnp.int32),
            pltpu.VMEM((be, 128), jnp.float32),
            pltpu.VMEM((be, 128), jnp.float32),
            pltpu.VMEM_SHARED((n_pad, 128), jnp.float32),
            pltpu.SemaphoreType.DMA,
            pltpu.SemaphoreType.DMA,
            pltpu.SemaphoreType.DMA,
            pltpu.SemaphoreType.DMA,
        ),
    )
    def kern(s2_hbm, t2_hbm, ex_hbm, h_hbm, hp_hbm,
             ex_v, sidx2, tidx2, gb_0, gb_1, gb_2, gb_3, ob_0, ob_1, hp_sh,
             gsem_0, gsem_1, gsem_2, gsem_3):
        cid = lax.axis_index("c")
        sid = lax.axis_index("s")
        wid = sid * NC + cid
        base = wid * chunk
        brow = wid * nb
        pltpu.sync_copy(ex_hbm.at[pl.ds(base, chunk)], ex_v)
        pltpu.sync_copy(s2_hbm.at[pl.ds(brow, nb)], sidx2)
        pltpu.sync_copy(t2_hbm.at[pl.ds(brow, nb)], tidx2)

        def zrow(r, c):
            for fi in range(f // L):
                ob_0[r, pl.ds(fi * L, L)] = jnp.zeros((L,), jnp.float32)
            return c

        lax.fori_loop(0, be, zrow, 0)
        off = sid * nslice
        for j in range(nslice // be):
            pltpu.sync_copy(ob_0, hp_sh.at[pl.ds(off + j * be, be)])
        plsc.subcore_barrier()

        bufs = ((gb_0, ob_0, gsem_0), (gb_1, ob_1, gsem_1),
                (gb_2, ob_0, gsem_2), (gb_3, ob_1, gsem_3))
        for ib, (gb, ob, gsem) in enumerate(bufs):
            pltpu.async_copy(h_hbm.at[tidx2.at[ib]], gb, gsem)

        def bbody(j, c):
            for ib, (gb, ob, gsem) in enumerate(bufs):
                bi = nbuf * j + ib
                pltpu.make_async_copy(h_hbm.at[tidx2.at[bi]], gb, gsem
                                      ).wait()

                def sgroup(g, c2):
                    ex16 = ex_v[pl.ds(bi * be + g * L, L)]
                    for r2 in range(L):
                        bc = _bcast_lane(ex16, r2)
                        r = g * L + r2
                        for fi in range(f // 32):
                            v16 = gb[r, pl.ds(fi * L, L)]
                            v32 = plsc.bitcast(v16, jnp.bfloat16)
                            ev, ov = plsc.unpack(
                                v32, format=plsc.PackFormat.INTERLEAVED)
                            ob[r, pl.ds(fi * 32, L)] = ev * bc
                            ob[r, pl.ds(fi * 32 + L, L)] = ov * bc
                    return c2

                lax.fori_loop(0, be // L, sgroup, 0)

                @pl.when(bi + nbuf < nb)
                def _():
                    pltpu.async_copy(h_hbm.at[tidx2.at[bi + nbuf]], gb, gsem)

                pltpu.sync_copy(ob, hp_sh.at[sidx2.at[bi]], add=True)
            return c

        lax.fori_loop(0, nb // nbuf, bbody, 0)
        plsc.subcore_barrier()
        pltpu.sync_copy(hp_sh.at[pl.ds(off, nslice)],
                        hp_hbm.at[cid, pl.ds(off, nslice)])

    return kern(src2d, tgt2d, ex_e, h64)


def kernel(x, edge_index, W, a, Wp, bp):
    n, f = x.shape
    e_true = edge_index.shape[1]
    W0 = W[0]
    avec = a[0, :, 0]
    a2d = jnp.stack([avec[:f], avec[f:]], axis=1)

    h, pq = _tc_pre(x, W0, a2d)

    n_pad = -(-n // (NS * BE)) * (NS * BE)
    epad = -(-e_true // (NW * BE)) * (NW * BE)
    p_pad = jnp.pad(pq[:, 0], (0, n_pad - n))
    q_pad = jnp.pad(pq[:, 1], (0, n_pad - n))
    src_p = jnp.pad(edge_index[0], (0, epad - e_true))
    tgt_p = jnp.pad(edge_index[1], (0, epad - e_true))
    src2d = src_p.reshape(-1, 64)
    tgt2d = tgt_p.reshape(-1, 64)

    den0, den1, ex_e = _sc_pass1(src_p, tgt_p, p_pad, q_pad,
                                 n_pad=n_pad, n_true=n, e_true=e_true)
    h64 = lax.bitcast_convert_type(h.reshape(n, f // 2, 2), jnp.int32)
    hp2 = _sc_pass2(src2d, tgt2d, ex_e, h64, n_pad=n_pad)
    blk = np.arange(0, f, 32)[:, None]
    half = np.concatenate([2 * np.arange(16), 2 * np.arange(16) + 1])[None, :]
    idx_list = jnp.asarray((blk + half).reshape(-1), dtype=jnp.int32)
    Wp_perm = Wp[:, idx_list]
    return _tc_post(hp2[0, :n], hp2[1, :n],
                    den0[:n].reshape(n, 1), den1[:n].reshape(n, 1),
                    n, Wp_perm, bp)
